# Initial kernel scaffold; baseline (speedup 1.0000x reference)
import jax
import jax.numpy as jnp
from jax.experimental import pallas as pl


def kernel(x, edge_index, edge_attr, p):
    raise NotImplementedError("write your pallas kernel here")


# speedup vs baseline: 12.2554x; 12.2554x over previous
import functools

import jax
import jax.numpy as jnp
from jax import lax
from jax.experimental import pallas as pl
from jax.experimental.pallas import tpu as pltpu

N = 512
E = 16384
BLK = 2048
NBLK = E // BLK


def _leaky(x, s):
    return jnp.where(x > 0, x, s * x)


def _gat_layer(h, src, dst, dstT, ea, emean, Wl, Wr, We, attf, bias, H,
               ml_s, al_s):
    D = Wl.shape[1]
    C = D // H
    xl = jnp.dot(h, Wl, preferred_element_type=jnp.float32)
    xr = jnp.dot(h, Wr, preferred_element_type=jnp.float32)
    iota_n = lax.broadcasted_iota(jnp.int32, (1, N), 1)
    iota_nc = lax.broadcasted_iota(jnp.int32, (N, 1), 0)
    G = (lax.broadcasted_iota(jnp.int32, (D, H), 0) // C
         == lax.broadcasted_iota(jnp.int32, (D, H), 1)).astype(jnp.float32)
    GT = (lax.broadcasted_iota(jnp.int32, (H, D), 0)
          == lax.broadcasted_iota(jnp.int32, (H, D), 1) // C).astype(jnp.float32)

    def alpha_of(m_pre):
        m = _leaky(m_pre, 0.2)
        return jnp.dot(m * attf, G, preferred_element_type=jnp.float32)

    def p1(i, gmax):
        sl = pl.ds(i * BLK, BLK)
        s_b = src[sl, :]
        d_b = dst[sl, :]
        oh_s = (s_b == iota_n).astype(jnp.float32)
        oh_d = (d_b == iota_n).astype(jnp.float32)
        ml = jnp.dot(oh_s, xl, preferred_element_type=jnp.float32)
        mr = jnp.dot(oh_d, xr, preferred_element_type=jnp.float32)
        m_pre = ml + mr + ea[sl, :] * We
        a = alpha_of(m_pre)
        ml_s[sl, :D] = ml
        al_s[sl, :H] = a
        return jnp.maximum(gmax, jnp.max(a, axis=0, keepdims=True))

    gmax = lax.fori_loop(0, NBLK, p1, jnp.full((1, H), -1e30, jnp.float32))
    m_loop = xl + xr + emean * We
    a_loop = alpha_of(m_loop)
    gmax = jnp.maximum(gmax, jnp.max(a_loop, axis=0, keepdims=True))

    e_loop = jnp.exp(a_loop - gmax)
    denom0 = e_loop
    out0 = jnp.dot(e_loop, GT, preferred_element_type=jnp.float32) * xl

    def p2(i, carry):
        denom, out = carry
        sl = pl.ds(i * BLK, BLK)
        d_bT = dstT[:, sl]
        oh_dT = (iota_nc == d_bT).astype(jnp.float32)
        e_b = jnp.exp(al_s[sl, :H] - gmax)
        denom = denom + jnp.dot(oh_dT, e_b, preferred_element_type=jnp.float32)
        w = jnp.dot(e_b, GT, preferred_element_type=jnp.float32) * ml_s[sl, :D]
        out = out + jnp.dot(oh_dT, w, preferred_element_type=jnp.float32)
        return denom, out

    denom, out = lax.fori_loop(0, NBLK, p2, (denom0, out0))
    denom_bc = jnp.dot(denom, GT, preferred_element_type=jnp.float32)
    return out / (denom_bc + 1e-16) + bias


def _layernorm(x, g, b):
    m = jnp.mean(x, axis=-1, keepdims=True)
    d = x - m
    v = jnp.mean(d * d, axis=-1, keepdims=True)
    return d * lax.rsqrt(v + 1e-5) * g + b


def _elu(x):
    return jnp.where(x > 0, x, jnp.exp(jnp.minimum(x, 0.0)) - 1.0)


def _gat_kernel(x, src, dst, dstT, ea,
                Wl1, Wr1, We1, att1, b1, g1, be1,
                Wl2, Wr2, We2, att2, b2, g2, be2,
                Wl3, Wr3, We3, att3, b3, g3, be3,
                W1top, W1bot, pb1,
                emb_o, A_o, B_o, ml_s, al_s):
    ea_v = ea[...]
    emean = jnp.sum(ea_v) * (1.0 / E)
    h = _gat_layer(x[...], src, dst, dstT, ea, emean, Wl1[...], Wr1[...],
                   We1[...], att1[...], b1[...], 4, ml_s, al_s)
    h = _elu(_layernorm(h, g1[...], be1[...]))
    h = _gat_layer(h, src, dst, dstT, ea, emean, Wl2[...], Wr2[...],
                   We2[...], att2[...], b2[...], 4, ml_s, al_s)
    h = _elu(_layernorm(h, g2[...], be2[...]))
    h = _gat_layer(h, src, dst, dstT, ea, emean, Wl3[...], Wr3[...],
                   We3[...], att3[...], b3[...], 1, ml_s, al_s)
    emb = _layernorm(h, g3[...], be3[...])
    emb_o[...] = emb
    A_o[...] = jnp.dot(emb, W1top[...], preferred_element_type=jnp.float32) + pb1[...]
    B_o[...] = jnp.dot(emb, W1bot[...], preferred_element_type=jnp.float32)


def _pair_kernel(A_blk, B_all, g_all, be_all, W2_all, b2_all, out_ref):
    I = A_blk.shape[0]
    t = A_blk[...][:, None, :] + B_all[...][None, :, :]
    t2 = t.reshape(I * N, 128)
    Gm = (lax.broadcasted_iota(jnp.int32, (128, 4), 0) // 32
          == lax.broadcasted_iota(jnp.int32, (128, 4), 1)).astype(jnp.float32)
    GmT = (lax.broadcasted_iota(jnp.int32, (4, 128), 0)
           == lax.broadcasted_iota(jnp.int32, (4, 128), 1) // 32).astype(jnp.float32)
    gs = jnp.dot(t2, Gm, preferred_element_type=jnp.float32)
    mean = gs * (1.0 / 32.0)
    d = t2 - jnp.dot(mean, GmT, preferred_element_type=jnp.float32)
    vs = jnp.dot(d * d, Gm, preferred_element_type=jnp.float32) * (1.0 / 32.0)
    rstd = lax.rsqrt(vs + 1e-5)
    hh = d * jnp.dot(rstd, GmT, preferred_element_type=jnp.float32)
    hh = hh * g_all[...] + be_all[...]
    hh = _leaky(hh, 0.1)
    s = jnp.dot(hh * W2_all[...], Gm, preferred_element_type=jnp.float32)
    s = s + b2_all[...]
    out_ref[...] = s.reshape(I, N, 4)


@jax.jit
def kernel(x, edge_index, edge_attr, p):
    src = edge_index[0].reshape(E, 1)
    dst = edge_index[1].reshape(E, 1)
    dstT = edge_index[1].reshape(1, E)
    r1 = lambda a: a.reshape(1, -1)
    W1top = jnp.concatenate([p['pd%d_W1' % i][:32] for i in range(4)], axis=1)
    W1bot = jnp.concatenate([p['pd%d_W1' % i][32:] for i in range(4)], axis=1)
    pb1 = jnp.concatenate([p['pd%d_b1' % i] for i in range(4)]).reshape(1, 128)
    g_all = jnp.concatenate([p['pd%d_g' % i] for i in range(4)]).reshape(1, 128)
    be_all = jnp.concatenate([p['pd%d_be' % i] for i in range(4)]).reshape(1, 128)
    W2_all = jnp.concatenate([p['pd%d_W2' % i][:, 0] for i in range(4)]).reshape(1, 128)
    b2_all = jnp.stack([p['pd%d_b2' % i][0] for i in range(4)]).reshape(1, 4)

    emb, A_all, B_all = pl.pallas_call(
        _gat_kernel,
        out_shape=[
            jax.ShapeDtypeStruct((N, 32), jnp.float32),
            jax.ShapeDtypeStruct((N, 128), jnp.float32),
            jax.ShapeDtypeStruct((N, 128), jnp.float32),
        ],
        scratch_shapes=[
            pltpu.VMEM((E, 64), jnp.float32),
            pltpu.VMEM((E, 4), jnp.float32),
        ],
    )(x, src, dst, dstT, edge_attr,
      p['Wl1'], p['Wr1'], r1(p['We1']), r1(p['att1']), r1(p['b1']),
      r1(p['ln1_g']), r1(p['ln1_b']),
      p['Wl2'], p['Wr2'], r1(p['We2']), r1(p['att2']), r1(p['b2']),
      r1(p['ln2_g']), r1(p['ln2_b']),
      p['Wl3'], p['Wr3'], r1(p['We3']), r1(p['att3']), r1(p['b3']),
      r1(p['ln3_g']), r1(p['ln3_b']),
      W1top, W1bot, pb1)

    I = 16
    scores = pl.pallas_call(
        _pair_kernel,
        grid=(N // I,),
        in_specs=[
            pl.BlockSpec((I, 128), lambda i: (i, 0)),
            pl.BlockSpec((N, 128), lambda i: (0, 0)),
            pl.BlockSpec((1, 128), lambda i: (0, 0)),
            pl.BlockSpec((1, 128), lambda i: (0, 0)),
            pl.BlockSpec((1, 128), lambda i: (0, 0)),
            pl.BlockSpec((1, 4), lambda i: (0, 0)),
        ],
        out_specs=pl.BlockSpec((I, N, 4), lambda i: (i, 0, 0)),
        out_shape=jax.ShapeDtypeStruct((N, N, 4), jnp.float32),
    )(A_all, B_all, g_all, be_all, W2_all, b2_all)

    return scores, emb
